# Initial kernel scaffold; baseline (speedup 1.0000x reference)
import jax
import jax.numpy as jnp
from jax.experimental import pallas as pl


def kernel(x, pe):
    raise NotImplementedError("write your pallas kernel here")


# speedup vs baseline: 4.8873x; 4.8873x over previous
import functools

import jax
import jax.numpy as jnp
from jax import lax
from jax.experimental import pallas as pl
from jax.experimental.pallas import tpu as pltpu
from jax.experimental.pallas import tpu_sc as plsc

_NC = 2
_NS = 16
_NW = _NC * _NS

_LANES = 128
_CH = 8


@functools.partial(jax.jit, static_argnums=(2, 3))
def _gather_rows(idx2d, table, n_idx_rows, d_model):
    rows_per_w = n_idx_rows // _NW
    n_chunks = rows_per_w // _CH
    b_total = n_idx_rows * _LANES

    mesh = plsc.VectorSubcoreMesh(core_axis_name="c", subcore_axis_name="s")

    @functools.partial(
        pl.kernel,
        mesh=mesh,
        compiler_params=pltpu.CompilerParams(use_tc_tiling_on_sc=False),
        out_type=jax.ShapeDtypeStruct((b_total, d_model), jnp.float32),
        scratch_types=[
            pltpu.VMEM((_CH, _LANES), jnp.int32),
            pltpu.VMEM((_CH * _LANES, d_model), jnp.float32),
            pltpu.SemaphoreType.DMA,
        ],
    )
    def k(table_hbm, idx_hbm, out_hbm, idx_v, rows_v, sem):
        wid = lax.axis_index("s") * _NC + lax.axis_index("c")
        row0 = wid * rows_per_w

        def body(g, carry):
            r = row0 + g * _CH
            pltpu.sync_copy(idx_hbm.at[pl.ds(r, _CH)], idx_v)
            copies = []
            for j in range(_CH):
                copies.append(
                    pltpu.async_copy(
                        table_hbm.at[idx_v.at[j]],
                        rows_v.at[pl.ds(j * _LANES, _LANES)],
                        sem,
                    )
                )
            for c in copies:
                c.wait()
            pltpu.sync_copy(rows_v, out_hbm.at[pl.ds(r * _LANES, _CH * _LANES)])
            return carry

        lax.fori_loop(0, n_chunks, body, 0)

    return k(table, idx2d)


def kernel(x, pe):
    n_rows, n_cols = x.shape
    d_model = pe.shape[1]
    b_total = n_rows * n_cols
    idx2d = x.reshape(b_total // _LANES, _LANES)
    out = _gather_rows(idx2d, pe, b_total // _LANES, d_model)
    return out.reshape(n_rows, n_cols, d_model)


# speedup vs baseline: 4.9613x; 1.0151x over previous
import functools

import jax
import jax.numpy as jnp
from jax import lax
from jax.experimental import pallas as pl
from jax.experimental.pallas import tpu as pltpu
from jax.experimental.pallas import tpu_sc as plsc

_NC = 2
_NS = 16
_NW = _NC * _NS

_LANES = 128
_CH = 4
_NBUF = 2


@functools.partial(jax.jit, static_argnums=(2, 3))
def _gather_rows(idx2d, table, n_idx_rows, d_model):
    rows_per_w = n_idx_rows // _NW
    n_chunks = rows_per_w // _CH
    b_total = n_idx_rows * _LANES

    mesh = plsc.VectorSubcoreMesh(core_axis_name="c", subcore_axis_name="s")

    @functools.partial(
        pl.kernel,
        mesh=mesh,
        compiler_params=pltpu.CompilerParams(use_tc_tiling_on_sc=False),
        out_type=jax.ShapeDtypeStruct((b_total, d_model), jnp.float32),
        scratch_types=[
            pltpu.VMEM((_NBUF, _CH, _LANES), jnp.int32),
            pltpu.VMEM((_NBUF, _CH * _LANES, d_model), jnp.float32),
            pltpu.SemaphoreType.DMA,
            pltpu.SemaphoreType.DMA,
            pltpu.SemaphoreType.DMA,
            pltpu.SemaphoreType.DMA,
            pltpu.SemaphoreType.DMA,
            pltpu.SemaphoreType.DMA,
        ],
    )
    def k(table_hbm, idx_hbm, out_hbm, idx_v, rows_v,
          sem_i0, sem_i1, sem_g0, sem_g1, sem_s0, sem_s1):
        sem_i = (sem_i0, sem_i1)
        sem_g = (sem_g0, sem_g1)
        sem_s = (sem_s0, sem_s1)
        wid = lax.axis_index("s") * _NC + lax.axis_index("c")
        row0 = wid * rows_per_w

        for b in range(_NBUF):
            pltpu.async_copy(
                idx_hbm.at[pl.ds(row0 + b * _CH, _CH)], idx_v.at[b], sem_i[b])

        def body(g, carry):
            for b in range(_NBUF):
                c = g * _NBUF + b
                r = row0 + c * _CH
                pltpu.make_async_copy(
                    idx_hbm.at[pl.ds(r, _CH)], idx_v.at[b], sem_i[b]).wait()
                @pl.when(g > 0)
                def _():
                    pltpu.make_async_copy(
                        rows_v.at[b],
                        out_hbm.at[pl.ds(r * _LANES, _CH * _LANES)],
                        sem_s[b]).wait()
                copies = []
                for j in range(_CH):
                    copies.append(pltpu.async_copy(
                        table_hbm.at[idx_v.at[b, j]],
                        rows_v.at[b, pl.ds(j * _LANES, _LANES)],
                        sem_g[b]))
                for cp in copies:
                    cp.wait()
                pltpu.async_copy(
                    rows_v.at[b],
                    out_hbm.at[pl.ds(r * _LANES, _CH * _LANES)],
                    sem_s[b])
                r_pre = row0 + jnp.minimum(c + _NBUF, n_chunks - 1) * _CH
                pltpu.async_copy(
                    idx_hbm.at[pl.ds(r_pre, _CH)], idx_v.at[b], sem_i[b])
            return carry

        lax.fori_loop(0, n_chunks // _NBUF, body, 0)

        for b in range(_NBUF):
            c = n_chunks - _NBUF + b
            r = row0 + c * _CH
            pltpu.make_async_copy(
                idx_hbm.at[pl.ds(r, _CH)], idx_v.at[b], sem_i[b]).wait()
            pltpu.make_async_copy(
                rows_v.at[b],
                out_hbm.at[pl.ds(r * _LANES, _CH * _LANES)],
                sem_s[b]).wait()

    return k(table, idx2d)


def kernel(x, pe):
    n_rows, n_cols = x.shape
    d_model = pe.shape[1]
    b_total = n_rows * n_cols
    idx2d = x.reshape(b_total // _LANES, _LANES)
    out = _gather_rows(idx2d, pe, b_total // _LANES, d_model)
    return out.reshape(n_rows, n_cols, d_model)


# speedup vs baseline: 4.9823x; 1.0042x over previous
import functools

import jax
import jax.numpy as jnp
from jax import lax
from jax.experimental import pallas as pl
from jax.experimental.pallas import tpu as pltpu
from jax.experimental.pallas import tpu_sc as plsc

_NC = 2
_NS = 16
_NW = _NC * _NS

_CI = 4
_NBUF = 2


@functools.partial(jax.jit, static_argnums=(2, 3, 4))
def _gather_rows(x, table, n_rows, n_cols, d_model):
    rows_per_w = n_rows // _NW
    n_chunks = rows_per_w // _CI

    segs = []
    off = 0
    while off < n_cols:
        seg = min(128, n_cols - off)
        segs.append((off, seg))
        off += seg

    mesh = plsc.VectorSubcoreMesh(core_axis_name="c", subcore_axis_name="s")

    @functools.partial(
        pl.kernel,
        mesh=mesh,
        compiler_params=pltpu.CompilerParams(use_tc_tiling_on_sc=False),
        out_type=jax.ShapeDtypeStruct((n_rows, n_cols, d_model), jnp.float32),
        scratch_types=[
            pltpu.VMEM((_NBUF, _CI, n_cols), jnp.int32),
            pltpu.VMEM((_NBUF, _CI, n_cols, d_model), jnp.float32),
            pltpu.SemaphoreType.DMA,
            pltpu.SemaphoreType.DMA,
            pltpu.SemaphoreType.DMA,
            pltpu.SemaphoreType.DMA,
            pltpu.SemaphoreType.DMA,
            pltpu.SemaphoreType.DMA,
        ],
    )
    def k(table_hbm, x_hbm, out_hbm, idx_v, rows_v,
          sem_i0, sem_i1, sem_g0, sem_g1, sem_s0, sem_s1):
        sem_i = (sem_i0, sem_i1)
        sem_g = (sem_g0, sem_g1)
        sem_s = (sem_s0, sem_s1)
        wid = lax.axis_index("s") * _NC + lax.axis_index("c")
        row0 = wid * rows_per_w

        for b in range(_NBUF):
            pltpu.async_copy(
                x_hbm.at[pl.ds(row0 + b * _CI, _CI)], idx_v.at[b], sem_i[b])

        def body(g, carry):
            for b in range(_NBUF):
                c = g * _NBUF + b
                r = row0 + c * _CI
                pltpu.make_async_copy(
                    x_hbm.at[pl.ds(r, _CI)], idx_v.at[b], sem_i[b]).wait()
                @pl.when(g > 0)
                def _():
                    pltpu.make_async_copy(
                        rows_v.at[b], out_hbm.at[pl.ds(r, _CI)],
                        sem_s[b]).wait()
                copies = []
                for ci in range(_CI):
                    for off, seg in segs:
                        copies.append(pltpu.async_copy(
                            table_hbm.at[idx_v.at[b, ci, pl.ds(off, seg)]],
                            rows_v.at[b, ci, pl.ds(off, seg)],
                            sem_g[b]))
                for cp in copies:
                    cp.wait()
                pltpu.async_copy(
                    rows_v.at[b], out_hbm.at[pl.ds(r, _CI)], sem_s[b])
                r_pre = row0 + jnp.minimum(c + _NBUF, n_chunks - 1) * _CI
                pltpu.async_copy(
                    x_hbm.at[pl.ds(r_pre, _CI)], idx_v.at[b], sem_i[b])
            return carry

        lax.fori_loop(0, n_chunks // _NBUF, body, 0)

        for b in range(_NBUF):
            c = n_chunks - _NBUF + b
            r = row0 + c * _CI
            pltpu.make_async_copy(
                x_hbm.at[pl.ds(r, _CI)], idx_v.at[b], sem_i[b]).wait()
            pltpu.make_async_copy(
                rows_v.at[b], out_hbm.at[pl.ds(r, _CI)], sem_s[b]).wait()

    return k(table, x)


def kernel(x, pe):
    n_rows, n_cols = x.shape
    d_model = pe.shape[1]
    return _gather_rows(x, pe, n_rows, n_cols, d_model)


# speedup vs baseline: 6.2852x; 1.2615x over previous
import functools

import jax
import jax.numpy as jnp
from jax import lax
from jax.experimental import pallas as pl
from jax.experimental.pallas import tpu as pltpu
from jax.experimental.pallas import tpu_sc as plsc

_NC = 2
_NS = 16
_NW = _NC * _NS

_L = 16
_IBW = 128
_KB = 8


@functools.partial(jax.jit, static_argnums=(2, 3, 4))
def _gather_phys(x_flat, tableT_flat, n_i, n_j, d_model):
    v_cap = tableT_flat.shape[0] // d_model
    n_kb = d_model // _KB
    n_ib = n_i // _IBW
    n_q = _NW // n_kb
    ib_per_q = n_ib // n_q
    blk = _IBW * n_j
    groups = _IBW // _L

    mesh = plsc.VectorSubcoreMesh(core_axis_name="c", subcore_axis_name="s")

    @functools.partial(
        pl.kernel,
        mesh=mesh,
        compiler_params=pltpu.CompilerParams(use_tc_tiling_on_sc=False,
                                             needs_layout_passes=False),
        out_type=jax.ShapeDtypeStruct((n_j, n_kb, n_ib, _KB, _IBW),
                                      jnp.float32),
        scratch_types=[
            pltpu.VMEM((_KB * v_cap,), jnp.float32),
            pltpu.VMEM((2, blk), jnp.int32),
            pltpu.VMEM((2, _KB, _IBW), jnp.float32),
            pltpu.SemaphoreType.DMA,
            pltpu.SemaphoreType.DMA,
            pltpu.SemaphoreType.DMA,
            pltpu.SemaphoreType.DMA,
        ],
    )
    def k(tableT_hbm, x_hbm, out_hbm, tbl_v, xblk_v, stg_v,
          sem_x0, sem_x1, sem_s0, sem_s1):
        sem_x = (sem_x0, sem_x1)
        sem_s = (sem_s0, sem_s1)
        wid = lax.axis_index("s") * _NC + lax.axis_index("c")
        kb = wid % n_kb
        q = wid // n_kb
        ib0 = q * ib_per_q

        pltpu.sync_copy(
            tableT_hbm.at[pl.ds(kb * (_KB * v_cap), _KB * v_cap)], tbl_v)

        il_off = [(lax.iota(jnp.int32, _L) + g * _L) * n_j
                  for g in range(groups)]
        kstep = jnp.full((_L,), v_cap, jnp.int32)

        for b in range(2):
            pltpu.async_copy(
                x_hbm.at[pl.ds((ib0 + b) * blk, blk)], xblk_v.at[b],
                sem_x[b])

        def ib_body(tp, carry):
            for bx in range(2):
                t = tp * 2 + bx
                ib = ib0 + t
                pltpu.make_async_copy(
                    x_hbm.at[pl.ds(ib * blk, blk)], xblk_v.at[bx],
                    sem_x[bx]).wait()

                def j_body(j2, carry2):
                    for bs in range(2):
                        j = j2 * 2 + bs
                        @pl.when(jnp.logical_or(t > 0, j2 > 0))
                        def _():
                            pltpu.make_async_copy(
                                stg_v.at[bs], out_hbm.at[j, kb, ib],
                                sem_s[bs]).wait()
                        jv = jnp.full((_L,), j, jnp.int32)
                        for g in range(groups):
                            v16 = plsc.load_gather(
                                xblk_v.at[bx], [il_off[g] + jv])
                            idx = v16
                            for kl in range(_KB):
                                row = plsc.load_gather(tbl_v, [idx])
                                stg_v[bs, kl, pl.ds(g * _L, _L)] = row
                                if kl + 1 < _KB:
                                    idx = idx + kstep
                        pltpu.async_copy(
                            stg_v.at[bs], out_hbm.at[j, kb, ib], sem_s[bs])
                    return carry2

                lax.fori_loop(0, n_j // 2, j_body, 0)

                t_pre = jnp.minimum(t + 2, ib_per_q - 1)
                pltpu.async_copy(
                    x_hbm.at[pl.ds((ib0 + t_pre) * blk, blk)],
                    xblk_v.at[bx], sem_x[bx])
            return carry

        lax.fori_loop(0, ib_per_q // 2, ib_body, 0)

        for b in range(2):
            pltpu.make_async_copy(
                x_hbm.at[pl.ds(ib0 * blk, blk)], xblk_v.at[b],
                sem_x[b]).wait()
            pltpu.make_async_copy(
                stg_v.at[b], out_hbm.at[n_j - 2 + b, kb, ib0 + ib_per_q - 1],
                sem_s[b]).wait()

    return k(tableT_flat, x_flat)


def kernel(x, pe):
    n_i, n_j = x.shape
    d_model = pe.shape[1]
    phys = _gather_phys(x.reshape(-1), pe.T.reshape(-1), n_i, n_j, d_model)
    return phys.transpose(2, 4, 0, 1, 3).reshape(n_i, n_j, d_model)
